# baseline (speedup 1.0000x reference)
import functools

import jax
import jax.numpy as jnp
from jax import lax
from jax.experimental import pallas as pl
from jax.experimental.pallas import tpu as pltpu
from jax.experimental.pallas import tpu_sc as plsc

B, LS, DM = 4096, 200, 64
DP = 128
TOT = B * LS
V = 1000000
NC, NS = 2, 16
NW = NC * NS
PER_W = TOT // NW
CHUNK = 512
NCHUNK = PER_W // CHUNK

RB = 128
NBLK = V // RB
TAIL = V - NBLK * RB

_mesh = plsc.VectorSubcoreMesh(core_axis_name="c", subcore_axis_name="s")


def _transpose_block(src_v, stg_v, nrow16, row0):
    lanes = lax.iota(jnp.int32, 16)

    def per_f(f, carry):
        def per_c(c, carry2):
            vals = src_v[f, pl.ds(row0 + c * 16, 16)]
            rows = c * 16 + lanes
            plsc.store_scatter(stg_v, [rows, jnp.full((16,), f, jnp.int32)], vals)
            plsc.store_scatter(
                stg_v, [rows, jnp.full((16,), f + DM, jnp.int32)], vals
            )
            return carry2

        return lax.fori_loop(0, nrow16, per_c, carry)

    lax.fori_loop(0, DM, per_f, 0)


@functools.partial(
    pl.kernel,
    out_type=jax.ShapeDtypeStruct((V, DP), jnp.float32),
    mesh=_mesh,
    scratch_types=[
        pltpu.VMEM((DM, RB), jnp.float32),
        pltpu.VMEM((RB, DP), jnp.float32),
        pltpu.VMEM((DM, TAIL), jnp.float32),
    ],
    compiler_params=pltpu.CompilerParams(needs_layout_passes=False),
)
def _sc_relayout(tt_hbm, tp_hbm, src_v, stg_v, tail_v):
    wid = lax.axis_index("s") * NC + lax.axis_index("c")
    nblk_w = (NBLK + NW - 1) // NW

    def body(i, carry):
        t = wid + i * NW

        @pl.when(t < NBLK)
        def _():
            off = pl.multiple_of(t * RB, RB)
            pltpu.sync_copy(tt_hbm.at[:, pl.ds(off, RB)], src_v)
            _transpose_block(src_v, stg_v, RB // 16, 0)
            pltpu.sync_copy(stg_v, tp_hbm.at[pl.ds(off, RB)])

        return carry

    lax.fori_loop(0, nblk_w, body, 0)

    @pl.when(wid == 0)
    def _():
        pltpu.sync_copy(tt_hbm.at[:, pl.ds(NBLK * RB, TAIL)], tail_v)
        _transpose_block(tail_v, stg_v, TAIL // 16, 0)
        pltpu.sync_copy(
            stg_v.at[pl.ds(0, TAIL)], tp_hbm.at[pl.ds(NBLK * RB, TAIL)]
        )


@functools.partial(
    pl.kernel,
    out_type=jax.ShapeDtypeStruct((TOT, DP), jnp.float32),
    mesh=_mesh,
    scratch_types=[
        pltpu.VMEM((PER_W,), jnp.int32),
        pltpu.VMEM((CHUNK, DP), jnp.float32),
        pltpu.SemaphoreType.DMA,
    ],
)
def _sc_gather(idx_hbm, table_hbm, out_hbm, idx_v, rows_v, gsem):
    wid = lax.axis_index("s") * NC + lax.axis_index("c")
    base = wid * PER_W
    pltpu.sync_copy(idx_hbm.at[pl.ds(base, PER_W)], idx_v)

    def body(i, carry):
        off = i * CHUNK
        pltpu.async_copy(
            table_hbm.at[idx_v.at[pl.ds(off, CHUNK)]], rows_v, gsem
        ).wait()
        pltpu.sync_copy(rows_v, out_hbm.at[pl.ds(base + off, CHUNK)])
        return carry

    lax.fori_loop(0, NCHUNK, body, 0)


def kernel(source, table):
    idx = source.T.reshape(TOT).astype(jnp.int32)
    tpack = _sc_relayout(table.T)
    out = _sc_gather(idx, tpack)
    return out.reshape(LS, B, DP)[:, :, :DM].transpose(1, 0, 2)
